# baseline (speedup 1.0000x reference)
import functools

import jax
import jax.numpy as jnp
from jax import lax
from jax.experimental import pallas as pl
from jax.experimental.pallas import tpu as pltpu
from jax.experimental.pallas import tpu_sc as plsc

B = 128
H = 8
V = 4096
NW = 32
TASKS = B * H
TPW = TASKS // NW
GROUPS = TPW // 16
LANE = 16


def _lane_perm(v, idx):
    return lax.gather(
        v, idx[:, None],
        dimension_numbers=lax.GatherDimensionNumbers(
            offset_dims=(), collapsed_slice_dims=(0,), start_index_map=(0,)),
        slice_sizes=(1,),
        mode=lax.GatherScatterMode.PROMISE_IN_BOUNDS)


def _allreduce(v, lanes, op):
    for k in (8, 4, 2, 1):
        v = op(v, _lane_perm(v, lanes ^ k))
    return v


def _sc_stats_kernel(policy_hbm, actions_hbm, out_hbm, buf, act_v, stage):
    wid = lax.axis_index("s") * 2 + lax.axis_index("c")
    head = wid // 4
    lanes = lax.iota(jnp.int32, LANE)

    for g in range(GROUPS):
        rbase = (wid % 4) * (GROUPS * LANE) + g * LANE
        pltpu.sync_copy(
            policy_hbm.at[pl.ds(rbase, LANE), pl.ds(head * V, V)], buf)
        pltpu.sync_copy(
            actions_hbm.at[pl.ds(head * B + rbase, LANE)], act_v)

        av = act_v[...]

        def task_body(t, carry):
            m_vec, s_vec, u_vec, xa_vec = carry
            at = _lane_perm(av, jnp.full((LANE,), t, jnp.int32))

            def max_body(j, acc):
                macc, xacc = acc
                v = buf[t, pl.ds(j * LANE, LANE)]
                cols = j * LANE + lanes
                xacc = jnp.where(cols == at, v, xacc)
                return jnp.maximum(macc, v), xacc

            macc, xacc = lax.fori_loop(
                0, V // LANE, max_body,
                (jnp.full((LANE,), -jnp.inf, dtype=jnp.float32),
                 jnp.zeros((LANE,), jnp.float32)))
            mt = _allreduce(macc, lanes, jnp.maximum)
            xat = _allreduce(xacc, lanes, jnp.add)

            def sum_body(j, acc):
                sacc, uacc = acc
                v = buf[t, pl.ds(j * LANE, LANE)]
                d = v - mt
                e = jnp.exp(d)
                return sacc + e, uacc + d * e

            sacc, uacc = lax.fori_loop(
                0, V // LANE, sum_body,
                (jnp.zeros((LANE,), jnp.float32),
                 jnp.zeros((LANE,), jnp.float32)))
            st = _allreduce(sacc, lanes, jnp.add)
            ut = _allreduce(uacc, lanes, jnp.add)

            sel = lanes == t
            m_vec = jnp.where(sel, mt, m_vec)
            s_vec = jnp.where(sel, st, s_vec)
            u_vec = jnp.where(sel, ut, u_vec)
            xa_vec = jnp.where(sel, xat, xa_vec)
            return m_vec, s_vec, u_vec, xa_vec

        zeros = jnp.zeros((LANE,), jnp.float32)
        m_vec, s_vec, u_vec, xa_vec = lax.fori_loop(
            0, LANE, task_body, (zeros, zeros, zeros, zeros))

        stage[0, :] = m_vec
        stage[1, :] = s_vec
        stage[2, :] = u_vec
        stage[3, :] = xa_vec
        for k in range(4):
            pltpu.sync_copy(
                stage.at[k], out_hbm.at[k, head, pl.ds(rbase, LANE)])


def _sc_stats(policy_output, actions_hm):
    mesh = plsc.VectorSubcoreMesh(core_axis_name="c", subcore_axis_name="s")
    k = functools.partial(
        pl.kernel,
        mesh=mesh,
        out_type=jax.ShapeDtypeStruct((4, H, B), jnp.float32),
        scratch_types=[
            pltpu.VMEM((LANE, V), jnp.float32),
            pltpu.VMEM((LANE,), jnp.int32),
            pltpu.VMEM((4, LANE), jnp.float32),
        ],
    )(_sc_stats_kernel)
    return k(policy_output, actions_hm)


def _combine_body(stats_ref, out_ref):
    m = stats_ref[0]
    s = stats_ref[1]
    u = stats_ref[2]
    xa = stats_ref[3]
    logs = jnp.log(s)
    lp = xa - m - logs
    ent = logs - u / s
    out_ref[0:1, :] = jnp.sum(lp, axis=0, keepdims=True)
    out_ref[1:2, :] = jnp.sum(ent, axis=0, keepdims=True)


def _combine(stats):
    return pl.pallas_call(
        _combine_body,
        out_shape=jax.ShapeDtypeStruct((2, B), jnp.float32),
    )(stats)


def kernel(policy_output, actions):
    actions_hm = actions.T.reshape(-1)
    stats = _sc_stats(policy_output, actions_hm)
    return _combine(stats)


# speedup vs baseline: 1.9487x; 1.9487x over previous
import functools

import jax
import jax.numpy as jnp
from jax import lax
from jax.experimental import pallas as pl
from jax.experimental.pallas import tpu as pltpu
from jax.experimental.pallas import tpu_sc as plsc

B = 128
H = 8
V = 4096
NW = 32
TASKS = B * H
TPW = TASKS // NW
GROUPS = TPW // 16
LANE = 16


def _lane_perm(v, idx):
    return lax.gather(
        v, idx[:, None],
        dimension_numbers=lax.GatherDimensionNumbers(
            offset_dims=(), collapsed_slice_dims=(0,), start_index_map=(0,)),
        slice_sizes=(1,),
        mode=lax.GatherScatterMode.PROMISE_IN_BOUNDS)


def _allreduce(v, lanes, op):
    for k in (8, 4, 2, 1):
        v = op(v, _lane_perm(v, lanes ^ k))
    return v


def _sc_stats_kernel(policy_hbm, actions_hbm, out_hbm, buf, act_v, stage):
    wid = lax.axis_index("s") * 2 + lax.axis_index("c")
    head = wid // 4
    lanes = lax.iota(jnp.int32, LANE)

    for g in range(GROUPS):
        rbase = (wid % 4) * (GROUPS * LANE) + g * LANE
        pltpu.sync_copy(
            policy_hbm.at[pl.ds(rbase, LANE), pl.ds(head * V, V)], buf)
        pltpu.sync_copy(
            actions_hbm.at[pl.ds(head * B + rbase, LANE)], act_v)

        U = 4
        av = act_v[...]

        def task_body(t, carry):
            m_vec, s_vec, u_vec, xa_vec = carry
            at = _lane_perm(av, jnp.full((LANE,), t, jnp.int32))

            def max_body(jj, state):
                accs, xacc, cols = state
                base = jj * (U * LANE)
                accs = list(accs)
                for k in range(U):
                    v = buf[t, pl.ds(base + k * LANE, LANE)]
                    ck = cols + (k * LANE)
                    xacc = jnp.where(ck == at, v, xacc)
                    accs[k] = jnp.maximum(accs[k], v)
                return tuple(accs), xacc, cols + (U * LANE)

            maccs, xacc, _ = lax.fori_loop(
                0, V // (U * LANE), max_body,
                (tuple(jnp.full((LANE,), -jnp.inf, jnp.float32)
                       for _ in range(U)),
                 jnp.zeros((LANE,), jnp.float32), lanes))
            macc = jnp.maximum(jnp.maximum(maccs[0], maccs[1]),
                               jnp.maximum(maccs[2], maccs[3]))
            mt = _allreduce(macc, lanes, jnp.maximum)
            xat = _allreduce(xacc, lanes, jnp.add)

            def sum_body(jj, accs):
                ss, uu = accs
                base = jj * (U * LANE)
                ss, uu = list(ss), list(uu)
                for k in range(U):
                    v = buf[t, pl.ds(base + k * LANE, LANE)]
                    d = v - mt
                    e = jnp.exp(d)
                    ss[k] = ss[k] + e
                    uu[k] = uu[k] + d * e
                return tuple(ss), tuple(uu)

            zeros4 = tuple(jnp.zeros((LANE,), jnp.float32) for _ in range(U))
            ss, uu = lax.fori_loop(
                0, V // (U * LANE), sum_body, (zeros4, zeros4))
            st = _allreduce((ss[0] + ss[1]) + (ss[2] + ss[3]), lanes, jnp.add)
            ut = _allreduce((uu[0] + uu[1]) + (uu[2] + uu[3]), lanes, jnp.add)

            sel = lanes == t
            m_vec = jnp.where(sel, mt, m_vec)
            s_vec = jnp.where(sel, st, s_vec)
            u_vec = jnp.where(sel, ut, u_vec)
            xa_vec = jnp.where(sel, xat, xa_vec)
            return m_vec, s_vec, u_vec, xa_vec

        zeros = jnp.zeros((LANE,), jnp.float32)
        m_vec, s_vec, u_vec, xa_vec = lax.fori_loop(
            0, LANE, task_body, (zeros, zeros, zeros, zeros))

        stage[0, :] = m_vec
        stage[1, :] = s_vec
        stage[2, :] = u_vec
        stage[3, :] = xa_vec
        for k in range(4):
            pltpu.sync_copy(
                stage.at[k], out_hbm.at[k, head, pl.ds(rbase, LANE)])


def _sc_stats(policy_output, actions_hm):
    mesh = plsc.VectorSubcoreMesh(core_axis_name="c", subcore_axis_name="s")
    k = functools.partial(
        pl.kernel,
        mesh=mesh,
        out_type=jax.ShapeDtypeStruct((4, H, B), jnp.float32),
        scratch_types=[
            pltpu.VMEM((LANE, V), jnp.float32),
            pltpu.VMEM((LANE,), jnp.int32),
            pltpu.VMEM((4, LANE), jnp.float32),
        ],
    )(_sc_stats_kernel)
    return k(policy_output, actions_hm)


def _combine_body(stats_ref, out_ref):
    m = stats_ref[0]
    s = stats_ref[1]
    u = stats_ref[2]
    xa = stats_ref[3]
    logs = jnp.log(s)
    lp = xa - m - logs
    ent = logs - u / s
    out_ref[0:1, :] = jnp.sum(lp, axis=0, keepdims=True)
    out_ref[1:2, :] = jnp.sum(ent, axis=0, keepdims=True)


def _combine(stats):
    return pl.pallas_call(
        _combine_body,
        out_shape=jax.ShapeDtypeStruct((2, B), jnp.float32),
    )(stats)


def kernel(policy_output, actions):
    actions_hm = actions.T.reshape(-1)
    stats = _sc_stats(policy_output, actions_hm)
    return _combine(stats)


# speedup vs baseline: 2.1351x; 1.0957x over previous
import functools

import jax
import jax.numpy as jnp
from jax import lax
from jax.experimental import pallas as pl
from jax.experimental.pallas import tpu as pltpu
from jax.experimental.pallas import tpu_sc as plsc

B = 128
H = 8
V = 4096
NW = 32
RPW = B // NW
LANE = 16
U = 4
LN2 = 0.6931471805599453


def _lane_perm(v, idx):
    return lax.gather(
        v, idx[:, None],
        dimension_numbers=lax.GatherDimensionNumbers(
            offset_dims=(), collapsed_slice_dims=(0,), start_index_map=(0,)),
        slice_sizes=(1,),
        mode=lax.GatherScatterMode.PROMISE_IN_BOUNDS)


def _allreduce(v, lanes, op, steps=(8, 4, 2, 1)):
    for k in steps:
        v = op(v, _lane_perm(v, lanes ^ k))
    return v


def _ln(s, lanes):
    bits = lax.bitcast_convert_type(s, jnp.int32)
    e = (bits >> 23) - 127
    f = lax.bitcast_convert_type((bits & 0x7FFFFF) | 0x3F800000,
                                 jnp.float32)
    t = f - 1.0
    z = t / (t + 2.0)
    z2 = z * z
    y = e.astype(jnp.float32) * LN2 + z * (2.0 + z2 * (2.0 / 3.0 + z2 * 0.4))
    for _ in range(2):
        y = y + (s * jnp.exp(-y) - 1.0)
    return y


def _row_stats(buf, av_g, parity, lanes, carry):
    m_vec, s_vec, u_vec, xa_vec = carry
    base_lane = parity * H

    def head_body(h, c):
        m_vec, s_vec, u_vec, xa_vec = c
        hbase = h * V
        at = _lane_perm(av_g, jnp.full((LANE,), base_lane + h, jnp.int32))
        fa = at & (LANE - 1)

        def max_body(jj, accs):
            maccs, xacc, cols = accs
            base = hbase + jj * (U * LANE)
            maccs = list(maccs)
            for k in range(U):
                v = buf[pl.ds(base + k * LANE, LANE)]
                xacc = jnp.where(cols + k * LANE == at, v, xacc)
                maccs[k] = jnp.maximum(maccs[k], v)
            return tuple(maccs), xacc, cols + U * LANE

        (maccs, xacc, _) = lax.fori_loop(
            0, V // (U * LANE), max_body,
            (tuple(jnp.full((LANE,), -jnp.inf, jnp.float32)
                   for _ in range(U)),
             jnp.zeros((LANE,), jnp.float32), lanes))
        macc = jnp.maximum(jnp.maximum(maccs[0], maccs[1]),
                           jnp.maximum(maccs[2], maccs[3]))
        mt = _allreduce(macc, lanes, jnp.maximum)
        xat = _lane_perm(xacc, fa)

        def sum_body(jj, accs):
            ss, uu = accs
            base = hbase + jj * (U * LANE)
            ss, uu = list(ss), list(uu)
            for k in range(U):
                v = buf[pl.ds(base + k * LANE, LANE)]
                d = v - mt
                ex = jnp.exp(d)
                ss[k] = ss[k] + ex
                uu[k] = uu[k] + d * ex
            return tuple(ss), tuple(uu)

        zeros4 = tuple(jnp.zeros((LANE,), jnp.float32) for _ in range(U))
        ss, uu = lax.fori_loop(0, V // (U * LANE), sum_body, (zeros4, zeros4))
        st = _allreduce((ss[0] + ss[1]) + (ss[2] + ss[3]), lanes, jnp.add)
        ut = _allreduce((uu[0] + uu[1]) + (uu[2] + uu[3]), lanes, jnp.add)

        sel = lanes == (base_lane + h)
        return (jnp.where(sel, mt, m_vec),
                jnp.where(sel, st, s_vec),
                jnp.where(sel, ut, u_vec),
                jnp.where(sel, xat, xa_vec))

    return lax.fori_loop(
        0, H, head_body, (m_vec, s_vec, u_vec, xa_vec))


def _sc_kernel(policy_hbm, actions_hbm, out_hbm, buf0, buf1, act_v,
               st_lp, st_ent, sem0, sem1):
    wid = lax.axis_index("s") * 2 + lax.axis_index("c")
    r0 = wid * RPW
    lanes = lax.iota(jnp.int32, LANE)

    pltpu.sync_copy(actions_hbm.at[pl.ds(wid * (RPW * H), RPW * H)], act_v)

    bufs = (buf0, buf1)
    sems = (sem0, sem1)
    copies = [None] * RPW
    for i in range(2):
        copies[i] = pltpu.async_copy(
            policy_hbm.at[r0 + i], bufs[i % 2], sems[i % 2])

    stage_lp = jnp.zeros((LANE,), jnp.float32)
    stage_ent = jnp.zeros((LANE,), jnp.float32)
    row_pick = (lanes & 1) * 8

    for g in range(RPW // 2):
        av_g = act_v[pl.ds(g * LANE, LANE)]
        zeros = jnp.zeros((LANE,), jnp.float32)
        carry = (zeros, zeros, zeros, zeros)
        for parity in range(2):
            i = g * 2 + parity
            copies[i].wait()
            carry = _row_stats(bufs[i % 2], av_g, parity, lanes, carry)
            if i + 2 < RPW:
                copies[i + 2] = pltpu.async_copy(
                    policy_hbm.at[r0 + i + 2], bufs[i % 2], sems[i % 2])
        m_vec, s_vec, u_vec, xa_vec = carry

        ln_s = _ln(s_vec, lanes)
        lp = xa_vec - m_vec - ln_s
        ent = ln_s - u_vec / s_vec
        lp = _allreduce(lp, lanes, jnp.add, steps=(4, 2, 1))
        ent = _allreduce(ent, lanes, jnp.add, steps=(4, 2, 1))
        gsel = (lanes >> 1) == g
        stage_lp = jnp.where(gsel, _lane_perm(lp, row_pick), stage_lp)
        stage_ent = jnp.where(gsel, _lane_perm(ent, row_pick), stage_ent)

    st_lp[...] = stage_lp
    st_ent[...] = stage_ent
    pltpu.sync_copy(st_lp, out_hbm.at[0, wid])
    pltpu.sync_copy(st_ent, out_hbm.at[1, wid])


def _sc_main(policy_output, actions_flat):
    mesh = plsc.VectorSubcoreMesh(core_axis_name="c", subcore_axis_name="s")
    k = functools.partial(
        pl.kernel,
        mesh=mesh,
        out_type=jax.ShapeDtypeStruct((2, NW, LANE), jnp.float32),
        scratch_types=[
            pltpu.VMEM((H * V,), jnp.float32),
            pltpu.VMEM((H * V,), jnp.float32),
            pltpu.VMEM((RPW * H,), jnp.int32),
            pltpu.VMEM((LANE,), jnp.float32),
            pltpu.VMEM((LANE,), jnp.float32),
            pltpu.SemaphoreType.DMA,
            pltpu.SemaphoreType.DMA,
        ],
    )(_sc_kernel)
    return k(policy_output, actions_flat)


def kernel(policy_output, actions):
    out = _sc_main(policy_output, actions.reshape(-1))
    return out[:, :, :RPW].reshape(2, B)


# speedup vs baseline: 2.2748x; 1.0654x over previous
import functools

import jax
import jax.numpy as jnp
from jax import lax
from jax.experimental import pallas as pl
from jax.experimental.pallas import tpu as pltpu
from jax.experimental.pallas import tpu_sc as plsc

B = 128
H = 8
V = 4096
NW = 32
RPW = B // NW
LANE = 16
U = 8
LN2 = 0.6931471805599453


def _lane_perm(v, idx):
    return lax.gather(
        v, idx[:, None],
        dimension_numbers=lax.GatherDimensionNumbers(
            offset_dims=(), collapsed_slice_dims=(0,), start_index_map=(0,)),
        slice_sizes=(1,),
        mode=lax.GatherScatterMode.PROMISE_IN_BOUNDS)


def _tree(vs, op):
    vs = list(vs)
    while len(vs) > 1:
        vs = [op(vs[i], vs[i + 1]) for i in range(0, len(vs) - 1, 2)] + (
            [vs[-1]] if len(vs) % 2 else [])
    return vs[0]


def _allreduce(v, lanes, op, steps=(8, 4, 2, 1)):
    for k in steps:
        v = op(v, _lane_perm(v, lanes ^ k))
    return v


def _ln(s, lanes):
    bits = lax.bitcast_convert_type(s, jnp.int32)
    e = (bits >> 23) - 127
    f = lax.bitcast_convert_type((bits & 0x7FFFFF) | 0x3F800000,
                                 jnp.float32)
    t = f - 1.0
    z = t / (t + 2.0)
    z2 = z * z
    y = e.astype(jnp.float32) * LN2 + z * (2.0 + z2 * (2.0 / 3.0 + z2 * 0.4))
    for _ in range(2):
        y = y + (s * jnp.exp(-y) - 1.0)
    return y


def _row_stats(buf, av_g, parity, lanes, carry):
    m_vec, s_vec, u_vec, xa_vec = carry
    base_lane = parity * H

    def head_body(h, c):
        m_vec, s_vec, u_vec, xa_vec = c
        hbase = h * V
        at = _lane_perm(av_g, jnp.full((LANE,), base_lane + h, jnp.int32))
        fa = at & (LANE - 1)

        def max_body(jj, accs):
            maccs, xacc, cols = accs
            base = hbase + jj * (U * LANE)
            maccs = list(maccs)
            for k in range(U):
                v = buf[pl.ds(base + k * LANE, LANE)]
                xacc = jnp.where(cols + k * LANE == at, v, xacc)
                maccs[k] = jnp.maximum(maccs[k], v)
            return tuple(maccs), xacc, cols + U * LANE

        (maccs, xacc, _) = lax.fori_loop(
            0, V // (U * LANE), max_body,
            (tuple(jnp.full((LANE,), -jnp.inf, jnp.float32)
                   for _ in range(U)),
             jnp.zeros((LANE,), jnp.float32), lanes))
        mt = _allreduce(_tree(maccs, jnp.maximum), lanes, jnp.maximum)
        xat = _lane_perm(xacc, fa)

        def sum_body(jj, accs):
            ss, uu = accs
            base = hbase + jj * (U * LANE)
            ss, uu = list(ss), list(uu)
            for k in range(U):
                v = buf[pl.ds(base + k * LANE, LANE)]
                d = v - mt
                ex = jnp.exp(d)
                ss[k] = ss[k] + ex
                uu[k] = uu[k] + d * ex
            return tuple(ss), tuple(uu)

        zeros4 = tuple(jnp.zeros((LANE,), jnp.float32) for _ in range(U))
        ss, uu = lax.fori_loop(0, V // (U * LANE), sum_body, (zeros4, zeros4))
        st = _allreduce(_tree(ss, jnp.add), lanes, jnp.add)
        ut = _allreduce(_tree(uu, jnp.add), lanes, jnp.add)

        sel = lanes == (base_lane + h)
        return (jnp.where(sel, mt, m_vec),
                jnp.where(sel, st, s_vec),
                jnp.where(sel, ut, u_vec),
                jnp.where(sel, xat, xa_vec))

    return lax.fori_loop(
        0, H, head_body, (m_vec, s_vec, u_vec, xa_vec))


def _sc_kernel(policy_hbm, actions_hbm, out_hbm, buf0, buf1, act_v,
               st_lp, st_ent, sem0, sem1):
    wid = lax.axis_index("s") * 2 + lax.axis_index("c")
    r0 = wid * RPW
    lanes = lax.iota(jnp.int32, LANE)

    pltpu.sync_copy(actions_hbm.at[pl.ds(wid * (RPW * H), RPW * H)], act_v)

    bufs = (buf0, buf1)
    sems = (sem0, sem1)
    copies = [None] * RPW
    for i in range(2):
        copies[i] = pltpu.async_copy(
            policy_hbm.at[r0 + i], bufs[i % 2], sems[i % 2])

    stage_lp = jnp.zeros((LANE,), jnp.float32)
    stage_ent = jnp.zeros((LANE,), jnp.float32)
    row_pick = (lanes & 1) * 8

    for g in range(RPW // 2):
        av_g = act_v[pl.ds(g * LANE, LANE)]
        zeros = jnp.zeros((LANE,), jnp.float32)
        carry = (zeros, zeros, zeros, zeros)
        for parity in range(2):
            i = g * 2 + parity
            copies[i].wait()
            carry = _row_stats(bufs[i % 2], av_g, parity, lanes, carry)
            if i + 2 < RPW:
                copies[i + 2] = pltpu.async_copy(
                    policy_hbm.at[r0 + i + 2], bufs[i % 2], sems[i % 2])
        m_vec, s_vec, u_vec, xa_vec = carry

        ln_s = _ln(s_vec, lanes)
        lp = xa_vec - m_vec - ln_s
        ent = ln_s - u_vec / s_vec
        lp = _allreduce(lp, lanes, jnp.add, steps=(4, 2, 1))
        ent = _allreduce(ent, lanes, jnp.add, steps=(4, 2, 1))
        gsel = (lanes >> 1) == g
        stage_lp = jnp.where(gsel, _lane_perm(lp, row_pick), stage_lp)
        stage_ent = jnp.where(gsel, _lane_perm(ent, row_pick), stage_ent)

    st_lp[...] = stage_lp
    st_ent[...] = stage_ent
    pltpu.sync_copy(st_lp, out_hbm.at[0, wid])
    pltpu.sync_copy(st_ent, out_hbm.at[1, wid])


def _sc_main(policy_output, actions_flat):
    mesh = plsc.VectorSubcoreMesh(core_axis_name="c", subcore_axis_name="s")
    k = functools.partial(
        pl.kernel,
        mesh=mesh,
        out_type=jax.ShapeDtypeStruct((2, NW, LANE), jnp.float32),
        scratch_types=[
            pltpu.VMEM((H * V,), jnp.float32),
            pltpu.VMEM((H * V,), jnp.float32),
            pltpu.VMEM((RPW * H,), jnp.int32),
            pltpu.VMEM((LANE,), jnp.float32),
            pltpu.VMEM((LANE,), jnp.float32),
            pltpu.SemaphoreType.DMA,
            pltpu.SemaphoreType.DMA,
        ],
    )(_sc_kernel)
    return k(policy_output, actions_flat)


def kernel(policy_output, actions):
    out = _sc_main(policy_output, actions.reshape(-1))
    return out[:, :, :RPW].reshape(2, B)


# speedup vs baseline: 2.7621x; 1.2142x over previous
import functools

import jax
import jax.numpy as jnp
from jax import lax
from jax.experimental import pallas as pl
from jax.experimental.pallas import tpu as pltpu
from jax.experimental.pallas import tpu_sc as plsc

B = 128
H = 8
V = 4096
NW = 32
R_SC = 64
RPW = R_SC // NW
LANE = 16
U = 8
LN2 = 0.6931471805599453


def _lane_perm(v, idx):
    return lax.gather(
        v, idx[:, None],
        dimension_numbers=lax.GatherDimensionNumbers(
            offset_dims=(), collapsed_slice_dims=(0,), start_index_map=(0,)),
        slice_sizes=(1,),
        mode=lax.GatherScatterMode.PROMISE_IN_BOUNDS)


def _tree(vs, op):
    vs = list(vs)
    while len(vs) > 1:
        vs = [op(vs[i], vs[i + 1]) for i in range(0, len(vs) - 1, 2)] + (
            [vs[-1]] if len(vs) % 2 else [])
    return vs[0]


def _allreduce(v, lanes, op, steps=(8, 4, 2, 1)):
    for k in steps:
        v = op(v, _lane_perm(v, lanes ^ k))
    return v


def _ln(s, lanes):
    bits = lax.bitcast_convert_type(s, jnp.int32)
    e = (bits >> 23) - 127
    f = lax.bitcast_convert_type((bits & 0x7FFFFF) | 0x3F800000,
                                 jnp.float32)
    t = f - 1.0
    z = t / (t + 2.0)
    z2 = z * z
    y = e.astype(jnp.float32) * LN2 + z * (2.0 + z2 * (2.0 / 3.0 + z2 * 0.4))
    for _ in range(2):
        y = y + (s * jnp.exp(-y) - 1.0)
    return y


def _row_stats(buf, av_g, parity, lanes, carry):
    m_vec, s_vec, u_vec, xa_vec = carry
    base_lane = parity * H

    def head_body(h, c):
        m_vec, s_vec, u_vec, xa_vec = c
        hbase = h * V
        at = _lane_perm(av_g, jnp.full((LANE,), base_lane + h, jnp.int32))
        fa = at & (LANE - 1)

        def max_body(jj, accs):
            maccs, xacc, cols = accs
            base = hbase + jj * (U * LANE)
            maccs = list(maccs)
            for k in range(U):
                v = buf[pl.ds(base + k * LANE, LANE)]
                xacc = jnp.where(cols + k * LANE == at, v, xacc)
                maccs[k] = jnp.maximum(maccs[k], v)
            return tuple(maccs), xacc, cols + U * LANE

        (maccs, xacc, _) = lax.fori_loop(
            0, V // (U * LANE), max_body,
            (tuple(jnp.full((LANE,), -jnp.inf, jnp.float32)
                   for _ in range(U)),
             jnp.zeros((LANE,), jnp.float32), lanes))
        mt = _allreduce(_tree(maccs, jnp.maximum), lanes, jnp.maximum)
        xat = _lane_perm(xacc, fa)

        def sum_body(jj, accs):
            ss, uu = accs
            base = hbase + jj * (U * LANE)
            ss, uu = list(ss), list(uu)
            for k in range(U):
                v = buf[pl.ds(base + k * LANE, LANE)]
                d = v - mt
                ex = jnp.exp(d)
                ss[k] = ss[k] + ex
                uu[k] = uu[k] + d * ex
            return tuple(ss), tuple(uu)

        zeros4 = tuple(jnp.zeros((LANE,), jnp.float32) for _ in range(U))
        ss, uu = lax.fori_loop(0, V // (U * LANE), sum_body, (zeros4, zeros4))
        st = _allreduce(_tree(ss, jnp.add), lanes, jnp.add)
        ut = _allreduce(_tree(uu, jnp.add), lanes, jnp.add)

        sel = lanes == (base_lane + h)
        return (jnp.where(sel, mt, m_vec),
                jnp.where(sel, st, s_vec),
                jnp.where(sel, ut, u_vec),
                jnp.where(sel, xat, xa_vec))

    return lax.fori_loop(
        0, H, head_body, (m_vec, s_vec, u_vec, xa_vec))


def _sc_kernel(policy_hbm, actions_hbm, out_hbm, buf0, buf1, act_v,
               st_lp, st_ent, sem0, sem1):
    wid = lax.axis_index("s") * 2 + lax.axis_index("c")
    r0 = wid * RPW
    lanes = lax.iota(jnp.int32, LANE)

    pltpu.sync_copy(actions_hbm.at[pl.ds(wid * (RPW * H), RPW * H)], act_v)

    bufs = (buf0, buf1)
    sems = (sem0, sem1)
    copies = [None] * RPW
    for i in range(2):
        copies[i] = pltpu.async_copy(
            policy_hbm.at[r0 + i], bufs[i % 2], sems[i % 2])

    stage_lp = jnp.zeros((LANE,), jnp.float32)
    stage_ent = jnp.zeros((LANE,), jnp.float32)
    row_pick = (lanes & 1) * 8

    for g in range(RPW // 2):
        av_g = act_v[pl.ds(g * LANE, LANE)]
        zeros = jnp.zeros((LANE,), jnp.float32)
        carry = (zeros, zeros, zeros, zeros)
        for parity in range(2):
            i = g * 2 + parity
            copies[i].wait()
            carry = _row_stats(bufs[i % 2], av_g, parity, lanes, carry)
            if i + 2 < RPW:
                copies[i + 2] = pltpu.async_copy(
                    policy_hbm.at[r0 + i + 2], bufs[i % 2], sems[i % 2])
        m_vec, s_vec, u_vec, xa_vec = carry

        ln_s = _ln(s_vec, lanes)
        lp = xa_vec - m_vec - ln_s
        ent = ln_s - u_vec / s_vec
        lp = _allreduce(lp, lanes, jnp.add, steps=(4, 2, 1))
        ent = _allreduce(ent, lanes, jnp.add, steps=(4, 2, 1))
        gsel = (lanes >> 1) == g
        stage_lp = jnp.where(gsel, _lane_perm(lp, row_pick), stage_lp)
        stage_ent = jnp.where(gsel, _lane_perm(ent, row_pick), stage_ent)

    st_lp[...] = stage_lp
    st_ent[...] = stage_ent
    pltpu.sync_copy(st_lp, out_hbm.at[0, wid])
    pltpu.sync_copy(st_ent, out_hbm.at[1, wid])


def _sc_main(policy_output, actions_flat):
    mesh = plsc.VectorSubcoreMesh(core_axis_name="c", subcore_axis_name="s")
    k = functools.partial(
        pl.kernel,
        mesh=mesh,
        out_type=jax.ShapeDtypeStruct((2, NW, LANE), jnp.float32),
        scratch_types=[
            pltpu.VMEM((H * V,), jnp.float32),
            pltpu.VMEM((H * V,), jnp.float32),
            pltpu.VMEM((RPW * H,), jnp.int32),
            pltpu.VMEM((LANE,), jnp.float32),
            pltpu.VMEM((LANE,), jnp.float32),
            pltpu.SemaphoreType.DMA,
            pltpu.SemaphoreType.DMA,
        ],
    )(_sc_kernel)
    return k(policy_output, actions_flat)


R_TC = B - R_SC


def _tc_body(x_ref, a_ref, out_ref):
    j = pl.program_id(0)
    x = x_ref[...]
    a = a_ref[0, 0, R_SC:][:, None]
    m = jnp.max(x, axis=1, keepdims=True)
    d = x - m
    e = jnp.exp(d)
    s = jnp.sum(e, axis=1)
    u = jnp.sum(d * e, axis=1)
    cols = lax.broadcasted_iota(jnp.int32, (R_TC, V), 1)
    xa = jnp.sum(jnp.where(cols == a, x, 0.0), axis=1)
    ln_s = jnp.log(s)
    lp = xa - m[:, 0] - ln_s
    ent = ln_s - u / s

    @pl.when(j == 0)
    def _():
        out_ref[...] = jnp.zeros_like(out_ref)

    out_ref[0, :] += lp
    out_ref[1, :] += ent


def _tc_rows(policy_output, actions_t3):
    return pl.pallas_call(
        _tc_body,
        grid=(H,),
        in_specs=[
            pl.BlockSpec((R_TC, V), lambda j: (R_SC // R_TC, j)),
            pl.BlockSpec((1, 1, B), lambda j: (j, 0, 0)),
        ],
        out_specs=pl.BlockSpec((2, R_TC), lambda j: (0, 0)),
        out_shape=jax.ShapeDtypeStruct((2, R_TC), jnp.float32),
    )(policy_output, actions_t3)


def kernel(policy_output, actions):
    sc = _sc_main(policy_output, actions[:R_SC].reshape(-1))
    tc = _tc_rows(policy_output, actions.T.reshape(H, 1, B))
    return jnp.concatenate([sc[:, :, :RPW].reshape(2, R_SC), tc], axis=1)
